# Initial kernel scaffold; baseline (speedup 1.0000x reference)
import jax
import jax.numpy as jnp
from jax.experimental import pallas as pl


def kernel(X, edge_index, edge_attr, W_root, W_rel, b_rel, W1, b1, W2, b2, W3, b3, W4, b4):
    raise NotImplementedError("write your pallas kernel here")


# speedup vs baseline: 1.9243x; 1.9243x over previous
import functools

import jax
import jax.numpy as jnp
from jax import lax
from jax.experimental import pallas as pl
from jax.experimental.pallas import tpu as pltpu
from jax.experimental.pallas import tpu_sc as plsc

NC = 2
NS = 16
NW = NC * NS
LANES = 16
CH = 100


def _lin2_body(x_ref, wa_ref, wb_ref, ya_ref, yb_ref):
    x = x_ref[...]
    ya_ref[...] = jnp.dot(x, wa_ref[...], preferred_element_type=jnp.float32)
    yb_ref[...] = jnp.dot(x, wb_ref[...], preferred_element_type=jnp.float32)


def _lin2(X, W_rel, W_root, bn):
    n, fin = X.shape
    h = W_rel.shape[1]
    grid = n // bn
    return pl.pallas_call(
        _lin2_body,
        grid=(grid,),
        in_specs=[
            pl.BlockSpec((bn, fin), lambda i: (i, 0)),
            pl.BlockSpec((fin, h), lambda i: (0, 0)),
            pl.BlockSpec((fin, h), lambda i: (0, 0)),
        ],
        out_specs=[
            pl.BlockSpec((bn, h), lambda i: (i, 0)),
            pl.BlockSpec((bn, h), lambda i: (i, 0)),
        ],
        out_shape=[
            jax.ShapeDtypeStruct((n, h), jnp.float32),
            jax.ShapeDtypeStruct((n, h), jnp.float32),
        ],
    )(X, W_rel, W_root)


def _sc_scatter(Y, src3, dst3, ea2, nch):
    n, h = Y.shape
    npad = -(-n // (NS * 8)) * (NS * 8)
    rpt = npad // NS
    zch = rpt // 5
    mesh = plsc.VectorSubcoreMesh(
        core_axis_name="c", subcore_axis_name="s", num_cores=NC, num_subcores=NS
    )

    @functools.partial(
        pl.kernel,
        out_type=jax.ShapeDtypeStruct((NC, npad, h), jnp.float32),
        mesh=mesh,
        compiler_params=pltpu.CompilerParams(needs_layout_passes=False, use_tc_tiling_on_sc=False),
        scratch_types=[
            pltpu.VMEM((nch, CH), jnp.int32),
            pltpu.VMEM((nch, CH), jnp.int32),
            pltpu.VMEM((nch * CH,), jnp.float32),
            pltpu.VMEM((CH, h), jnp.float32),
            pltpu.VMEM((zch, h), jnp.float32),
            pltpu.VMEM_SHARED((npad, h), jnp.float32),
            pltpu.SemaphoreType.DMA,
        ],
    )
    def run(y_hbm, src_hbm, dst_hbm, ea_hbm, out_hbm,
            src_v, dst_v, ea_v, rows_v, z_v, t_sh, sem):
        cid = lax.axis_index("c")
        sid = lax.axis_index("s")
        wid = sid * NC + cid
        pltpu.sync_copy(src_hbm.at[wid], src_v)
        pltpu.sync_copy(dst_hbm.at[wid], dst_v)
        pltpu.sync_copy(ea_hbm.at[wid], ea_v)

        zero = jnp.zeros((LANES,), jnp.float32)
        for r in range(zch):
            for c0 in range(h // LANES):
                z_v[r, pl.ds(c0 * LANES, LANES)] = zero
        for k in range(5):
            pltpu.sync_copy(z_v, t_sh.at[pl.ds(sid * rpt + k * zch, zch)])
        plsc.subcore_barrier()

        def chunk(j, carry):
            pltpu.async_copy(y_hbm.at[src_v.at[j]], rows_v, sem).wait()
            base = j * CH
            for e in range(CH):
                eab = plsc.load_gather(
                    ea_v, [jnp.full((LANES,), base + e, jnp.int32)])
                for c0 in range(h // LANES):
                    sl = pl.ds(c0 * LANES, LANES)
                    rows_v[e, sl] = rows_v[e, sl] * eab
            pltpu.sync_copy(rows_v, t_sh.at[dst_v.at[j]], add=True)
            return carry

        lax.fori_loop(0, nch, chunk, 0)
        plsc.subcore_barrier()
        pltpu.sync_copy(t_sh.at[pl.ds(sid * rpt, rpt)],
                        out_hbm.at[cid, pl.ds(sid * rpt, rpt)])

    return run(Y, src3, dst3, ea2)


def _mlp_body(tp_ref, r_ref, brel_ref, w1_ref, b1_ref, w2_ref, b2_ref,
              w3_ref, b3_ref, w4_ref, b4_ref, s_ref, ss_ref):
    h = tp_ref[0] + tp_ref[1] + r_ref[...] + brel_ref[...]
    z = jnp.maximum(
        jnp.dot(h, w1_ref[...], preferred_element_type=jnp.float32)
        + b1_ref[...], 0.0)
    z = jnp.maximum(
        jnp.dot(z, w2_ref[...], preferred_element_type=jnp.float32)
        + b2_ref[...], 0.0)
    z = jnp.maximum(
        jnp.dot(z, w3_ref[...], preferred_element_type=jnp.float32)
        + b3_ref[...], 0.0)
    s = jnp.dot(z, w4_ref[...], preferred_element_type=jnp.float32) + b4_ref[...]
    m = jnp.max(s, axis=-1, keepdims=True)
    e = jnp.exp(s - m)
    sm = e / jnp.sum(e, axis=-1, keepdims=True)
    s_ref[...] = sm
    ss_ref[...] = lax.dot_general(
        sm, sm, (((0,), (0,)), ((), ())),
        preferred_element_type=jnp.float32)[None]


def _mlp(t_part, R, b_rel, W1, b1, W2, b2, W3, b3, W4, b4, bn):
    n, h = R.shape
    c = W4.shape[1]
    grid = n // bn
    wfull = lambda shp: pl.BlockSpec(shp, lambda i: tuple(0 for _ in shp))
    return pl.pallas_call(
        _mlp_body,
        grid=(grid,),
        in_specs=[
            pl.BlockSpec((NC, bn, h), lambda i: (0, i, 0)),
            pl.BlockSpec((bn, h), lambda i: (i, 0)),
            wfull((1, h)),
            wfull(W1.shape), wfull((1, W1.shape[1])),
            wfull(W2.shape), wfull((1, W2.shape[1])),
            wfull(W3.shape), wfull((1, W3.shape[1])),
            wfull(W4.shape), wfull((1, W4.shape[1])),
        ],
        out_specs=[
            pl.BlockSpec((bn, c), lambda i: (i, 0)),
            pl.BlockSpec((1, c, c), lambda i: (i, 0, 0)),
        ],
        out_shape=[
            jax.ShapeDtypeStruct((n, c), jnp.float32),
            jax.ShapeDtypeStruct((grid, c, c), jnp.float32),
        ],
    )(t_part, R, b_rel.reshape(1, h), W1, b1.reshape(1, -1), W2,
      b2.reshape(1, -1), W3, b3.reshape(1, -1), W4, b4.reshape(1, -1))


def _sc_edge_losses(s_sm, src2, dst2, ea2, nch):
    n, c = s_sm.shape
    mesh = plsc.VectorSubcoreMesh(
        core_axis_name="c", subcore_axis_name="s", num_cores=NC, num_subcores=NS
    )
    epw = nch * CH
    per_vreg = LANES // c

    @functools.partial(
        pl.kernel,
        out_type=jax.ShapeDtypeStruct((NW, 2 * LANES), jnp.float32),
        mesh=mesh,
        compiler_params=pltpu.CompilerParams(needs_layout_passes=False, use_tc_tiling_on_sc=False),
        scratch_types=[
            pltpu.VMEM((n, c), jnp.float32),
            pltpu.VMEM((epw,), jnp.int32),
            pltpu.VMEM((epw,), jnp.int32),
            pltpu.VMEM((epw,), jnp.float32),
            pltpu.VMEM((2 * LANES,), jnp.float32),
            pltpu.SemaphoreType.DMA,
        ],
    )
    def run(s_hbm, src_hbm, dst_hbm, ea_hbm, out_hbm,
            s_v, src_v, dst_v, ea_v, buf_v, sem):
        cid = lax.axis_index("c")
        sid = lax.axis_index("s")
        wid = sid * NC + cid
        pltpu.sync_copy(s_hbm, s_v)
        pltpu.sync_copy(src_hbm.at[wid], src_v)
        pltpu.sync_copy(dst_hbm.at[wid], dst_v)
        pltpu.sync_copy(ea_hbm.at[wid], ea_v)

        lane = lax.iota(jnp.int32, LANES)
        roff = lane // c
        col = lane % c

        def chunk(j, acc):
            accn, accd = acc
            for v in range(CH // per_vreg):
                idx = jnp.full((LANES,), j * CH + v * per_vreg, jnp.int32) + roff
                srow = plsc.load_gather(src_v, [idx])
                drow = plsc.load_gather(dst_v, [idx])
                eav = plsc.load_gather(ea_v, [idx])
                sv = plsc.load_gather(s_v, [srow, col])
                dv = plsc.load_gather(s_v, [drow, col])
                es = eav * sv
                accn = accn + es * dv
                accd = accd + es * sv
            return (accn, accd)

        z = jnp.zeros((LANES,), jnp.float32)
        accn, accd = lax.fori_loop(0, nch, chunk, (z, z))
        buf_v[pl.ds(0, LANES)] = accn
        buf_v[pl.ds(LANES, LANES)] = accd
        pltpu.sync_copy(buf_v, out_hbm.at[wid])

    return run(s_sm, src2, dst2, ea2)


def _combine_body(nd_ref, ss_ref, mc_ref, or_ref):
    nd = nd_ref[...]
    num = jnp.sum(nd[:, :LANES])
    den = jnp.sum(nd[:, LANES:])
    mc_ref[...] = jnp.reshape(-(num / den), (1, 1))
    ss = jnp.sum(ss_ref[...], axis=0)
    k = ss.shape[0]
    ssn = jnp.sqrt(jnp.sum(ss * ss))
    eye = (lax.broadcasted_iota(jnp.int32, (k, k), 0)
           == lax.broadcasted_iota(jnp.int32, (k, k), 1)).astype(jnp.float32)
    diff = ss / ssn - eye / jnp.sqrt(jnp.float32(k))
    or_ref[...] = jnp.reshape(jnp.sqrt(jnp.sum(diff * diff)), (1, 1))


def _combine(nd, ss_part):
    g, c, _ = ss_part.shape
    return pl.pallas_call(
        _combine_body,
        in_specs=[
            pl.BlockSpec(nd.shape, lambda: (0, 0)),
            pl.BlockSpec(ss_part.shape, lambda: (0, 0, 0)),
        ],
        out_specs=[
            pl.BlockSpec((1, 1), lambda: (0, 0)),
            pl.BlockSpec((1, 1), lambda: (0, 0)),
        ],
        out_shape=[
            jax.ShapeDtypeStruct((1, 1), jnp.float32),
            jax.ShapeDtypeStruct((1, 1), jnp.float32),
        ],
    )(nd, ss_part)


def kernel(X, edge_index, edge_attr, W_root, W_rel, b_rel,
           W1, b1, W2, b2, W3, b3, W4, b4):
    n, fin = X.shape
    e = edge_attr.shape[0]
    h = W_rel.shape[1]

    nch = -(-e // (NW * CH))
    epad = NW * CH * nch
    src = edge_index[0]
    dst = edge_index[1]
    ea = edge_attr.astype(jnp.float32)
    if epad != e:
        pad = epad - e
        zi = jnp.zeros((pad,), jnp.int32)
        src = jnp.concatenate([src, zi])
        dst = jnp.concatenate([dst, zi])
        ea = jnp.concatenate([ea, jnp.zeros((pad,), jnp.float32)])
    src3 = src.reshape(NW, nch, CH)
    dst3 = dst.reshape(NW, nch, CH)
    ea2 = ea.reshape(NW, nch * CH)

    Y, R = _lin2(X, W_rel, W_root, bn=1000)
    t_part = _sc_scatter(Y, src3, dst3, ea2, nch)[:, :n, :]
    s_sm, ss_part = _mlp(t_part, R, b_rel, W1, b1, W2, b2, W3, b3, W4, b4,
                         bn=1000)
    if True:
        sv = s_sm[src]
        dv = s_sm[dst]
        num = jnp.sum(ea * jnp.sum(sv * dv, axis=-1))
        den = jnp.sum(ea * jnp.sum(sv * sv, axis=-1))
        nd = jnp.zeros((NW, 2 * LANES), jnp.float32)
        nd = nd.at[0, 0].set(num).at[0, LANES].set(den)
    else:
        nd = _sc_edge_losses(s_sm, src.reshape(NW, nch * CH),
                             dst.reshape(NW, nch * CH), ea2, nch)
    mc, orl = _combine(nd, ss_part)
    return (s_sm[None], mc.reshape(()), orl.reshape(()))


# speedup vs baseline: 8.7222x; 4.5327x over previous
import functools

import jax
import jax.numpy as jnp
from jax import lax
from jax.experimental import pallas as pl
from jax.experimental.pallas import tpu as pltpu
from jax.experimental.pallas import tpu_sc as plsc

NC = 2
NS = 16
NW = NC * NS
LANES = 16
CH = 100


def _lin2_body(x_ref, wa_ref, wb_ref, ya_ref, yb_ref):
    x = x_ref[...]
    ya_ref[...] = jnp.dot(x, wa_ref[...], preferred_element_type=jnp.float32)
    yb_ref[...] = jnp.dot(x, wb_ref[...], preferred_element_type=jnp.float32)


def _lin2(X, W_rel, W_root, bn):
    n, fin = X.shape
    h = W_rel.shape[1]
    grid = n // bn
    return pl.pallas_call(
        _lin2_body,
        grid=(grid,),
        in_specs=[
            pl.BlockSpec((bn, fin), lambda i: (i, 0)),
            pl.BlockSpec((fin, h), lambda i: (0, 0)),
            pl.BlockSpec((fin, h), lambda i: (0, 0)),
        ],
        out_specs=[
            pl.BlockSpec((bn, h), lambda i: (i, 0)),
            pl.BlockSpec((bn, h), lambda i: (i, 0)),
        ],
        out_shape=[
            jax.ShapeDtypeStruct((n, h), jnp.float32),
            jax.ShapeDtypeStruct((n, h), jnp.float32),
        ],
    )(X, W_rel, W_root)


def _sc_scatter(Y, src3, dst3, ea2, nch):
    n, h = Y.shape
    npad = -(-n // (NS * 8)) * (NS * 8)
    rpt = npad // NS
    zch = rpt // 5
    mesh = plsc.VectorSubcoreMesh(
        core_axis_name="c", subcore_axis_name="s", num_cores=NC, num_subcores=NS
    )

    @functools.partial(
        pl.kernel,
        out_type=jax.ShapeDtypeStruct((NC, npad, h), jnp.float32),
        mesh=mesh,
        compiler_params=pltpu.CompilerParams(needs_layout_passes=False, use_tc_tiling_on_sc=False),
        scratch_types=[
            pltpu.VMEM((nch, CH), jnp.int32),
            pltpu.VMEM((nch, CH), jnp.int32),
            pltpu.VMEM((nch * CH,), jnp.float32),
            pltpu.VMEM((CH, h), jnp.float32),
            pltpu.VMEM((zch, h), jnp.float32),
            pltpu.VMEM_SHARED((npad, h), jnp.float32),
            pltpu.SemaphoreType.DMA,
        ],
    )
    def run(y_hbm, src_hbm, dst_hbm, ea_hbm, out_hbm,
            src_v, dst_v, ea_v, rows_v, z_v, t_sh, sem):
        cid = lax.axis_index("c")
        sid = lax.axis_index("s")
        wid = sid * NC + cid
        pltpu.sync_copy(src_hbm.at[wid], src_v)
        pltpu.sync_copy(dst_hbm.at[wid], dst_v)
        pltpu.sync_copy(ea_hbm.at[wid], ea_v)

        zero = jnp.zeros((LANES,), jnp.float32)
        for r in range(zch):
            for c0 in range(h // LANES):
                z_v[r, pl.ds(c0 * LANES, LANES)] = zero
        for k in range(5):
            pltpu.sync_copy(z_v, t_sh.at[pl.ds(sid * rpt + k * zch, zch)])
        plsc.subcore_barrier()

        def chunk(j, carry):
            pltpu.async_copy(y_hbm.at[src_v.at[j]], rows_v, sem).wait()
            base = j * CH
            for e in range(CH):
                eab = plsc.load_gather(
                    ea_v, [jnp.full((LANES,), base + e, jnp.int32)])
                for c0 in range(h // LANES):
                    sl = pl.ds(c0 * LANES, LANES)
                    rows_v[e, sl] = rows_v[e, sl] * eab
            pltpu.sync_copy(rows_v, t_sh.at[dst_v.at[j]], add=True)
            return carry

        lax.fori_loop(0, nch, chunk, 0)
        plsc.subcore_barrier()
        pltpu.sync_copy(t_sh.at[pl.ds(sid * rpt, rpt)],
                        out_hbm.at[cid, pl.ds(sid * rpt, rpt)])

    return run(Y, src3, dst3, ea2)


def _mlp_body(tp_ref, r_ref, brel_ref, w1_ref, b1_ref, w2_ref, b2_ref,
              w3_ref, b3_ref, w4_ref, b4_ref, s_ref, ss_ref):
    h = tp_ref[0] + tp_ref[1] + r_ref[...] + brel_ref[...]
    z = jnp.maximum(
        jnp.dot(h, w1_ref[...], preferred_element_type=jnp.float32)
        + b1_ref[...], 0.0)
    z = jnp.maximum(
        jnp.dot(z, w2_ref[...], preferred_element_type=jnp.float32)
        + b2_ref[...], 0.0)
    z = jnp.maximum(
        jnp.dot(z, w3_ref[...], preferred_element_type=jnp.float32)
        + b3_ref[...], 0.0)
    s = jnp.dot(z, w4_ref[...], preferred_element_type=jnp.float32) + b4_ref[...]
    m = jnp.max(s, axis=-1, keepdims=True)
    e = jnp.exp(s - m)
    sm = e / jnp.sum(e, axis=-1, keepdims=True)
    s_ref[...] = sm
    ss_ref[...] = lax.dot_general(
        sm, sm, (((0,), (0,)), ((), ())),
        preferred_element_type=jnp.float32)[None]


def _mlp(t_part, R, b_rel, W1, b1, W2, b2, W3, b3, W4, b4, bn):
    n, h = R.shape
    c = W4.shape[1]
    grid = n // bn
    wfull = lambda shp: pl.BlockSpec(shp, lambda i: tuple(0 for _ in shp))
    return pl.pallas_call(
        _mlp_body,
        grid=(grid,),
        in_specs=[
            pl.BlockSpec((NC, bn, h), lambda i: (0, i, 0)),
            pl.BlockSpec((bn, h), lambda i: (i, 0)),
            wfull((1, h)),
            wfull(W1.shape), wfull((1, W1.shape[1])),
            wfull(W2.shape), wfull((1, W2.shape[1])),
            wfull(W3.shape), wfull((1, W3.shape[1])),
            wfull(W4.shape), wfull((1, W4.shape[1])),
        ],
        out_specs=[
            pl.BlockSpec((bn, c), lambda i: (i, 0)),
            pl.BlockSpec((1, c, c), lambda i: (i, 0, 0)),
        ],
        out_shape=[
            jax.ShapeDtypeStruct((n, c), jnp.float32),
            jax.ShapeDtypeStruct((grid, c, c), jnp.float32),
        ],
    )(t_part, R, b_rel.reshape(1, h), W1, b1.reshape(1, -1), W2,
      b2.reshape(1, -1), W3, b3.reshape(1, -1), W4, b4.reshape(1, -1))


def _sc_edge_losses(s_sm, src2, dst2, ea2, nch):
    n, c = s_sm.shape
    mesh = plsc.VectorSubcoreMesh(
        core_axis_name="c", subcore_axis_name="s", num_cores=NC, num_subcores=NS
    )
    epw = nch * CH
    per_vreg = LANES // c

    @functools.partial(
        pl.kernel,
        out_type=jax.ShapeDtypeStruct((NW, 2 * LANES), jnp.float32),
        mesh=mesh,
        compiler_params=pltpu.CompilerParams(needs_layout_passes=False, use_tc_tiling_on_sc=False),
        scratch_types=[
            pltpu.VMEM((n, c), jnp.float32),
            pltpu.VMEM((epw,), jnp.int32),
            pltpu.VMEM((epw,), jnp.int32),
            pltpu.VMEM((epw,), jnp.float32),
            pltpu.VMEM((2 * LANES,), jnp.float32),
            pltpu.SemaphoreType.DMA,
        ],
    )
    def run(s_hbm, src_hbm, dst_hbm, ea_hbm, out_hbm,
            s_v, src_v, dst_v, ea_v, buf_v, sem):
        cid = lax.axis_index("c")
        sid = lax.axis_index("s")
        wid = sid * NC + cid
        pltpu.sync_copy(s_hbm, s_v)
        pltpu.sync_copy(src_hbm.at[wid], src_v)
        pltpu.sync_copy(dst_hbm.at[wid], dst_v)
        pltpu.sync_copy(ea_hbm.at[wid], ea_v)

        lane = lax.iota(jnp.int32, LANES)
        roff = lane // c
        col = lane % c

        def chunk(j, acc):
            accn, accd = acc
            for v in range(CH // per_vreg):
                idx = jnp.full((LANES,), j * CH + v * per_vreg, jnp.int32) + roff
                srow = plsc.load_gather(src_v, [idx])
                drow = plsc.load_gather(dst_v, [idx])
                eav = plsc.load_gather(ea_v, [idx])
                sv = plsc.load_gather(s_v, [srow, col])
                dv = plsc.load_gather(s_v, [drow, col])
                es = eav * sv
                accn = accn + es * dv
                accd = accd + es * sv
            return (accn, accd)

        z = jnp.zeros((LANES,), jnp.float32)
        accn, accd = lax.fori_loop(0, nch, chunk, (z, z))
        buf_v[pl.ds(0, LANES)] = accn
        buf_v[pl.ds(LANES, LANES)] = accd
        pltpu.sync_copy(buf_v, out_hbm.at[wid])

    return run(s_sm, src2, dst2, ea2)


def _combine_body(nd_ref, ss_ref, mc_ref, or_ref):
    nd = nd_ref[...]
    num = jnp.sum(nd[:, :LANES])
    den = jnp.sum(nd[:, LANES:])
    mc_ref[...] = jnp.reshape(-(num / den), (1, 1))
    ss = jnp.sum(ss_ref[...], axis=0)
    k = ss.shape[0]
    ssn = jnp.sqrt(jnp.sum(ss * ss))
    eye = (lax.broadcasted_iota(jnp.int32, (k, k), 0)
           == lax.broadcasted_iota(jnp.int32, (k, k), 1)).astype(jnp.float32)
    diff = ss / ssn - eye / jnp.sqrt(jnp.float32(k))
    or_ref[...] = jnp.reshape(jnp.sqrt(jnp.sum(diff * diff)), (1, 1))


def _combine(nd, ss_part):
    g, c, _ = ss_part.shape
    return pl.pallas_call(
        _combine_body,
        in_specs=[
            pl.BlockSpec(nd.shape, lambda: (0, 0)),
            pl.BlockSpec(ss_part.shape, lambda: (0, 0, 0)),
        ],
        out_specs=[
            pl.BlockSpec((1, 1), lambda: (0, 0)),
            pl.BlockSpec((1, 1), lambda: (0, 0)),
        ],
        out_shape=[
            jax.ShapeDtypeStruct((1, 1), jnp.float32),
            jax.ShapeDtypeStruct((1, 1), jnp.float32),
        ],
    )(nd, ss_part)


def kernel(X, edge_index, edge_attr, W_root, W_rel, b_rel,
           W1, b1, W2, b2, W3, b3, W4, b4):
    n, fin = X.shape
    e = edge_attr.shape[0]
    h = W_rel.shape[1]

    nch = -(-e // (NW * CH))
    epad = NW * CH * nch
    src = edge_index[0]
    dst = edge_index[1]
    ea = edge_attr.astype(jnp.float32)
    if epad != e:
        pad = epad - e
        zi = jnp.zeros((pad,), jnp.int32)
        src = jnp.concatenate([src, zi])
        dst = jnp.concatenate([dst, zi])
        ea = jnp.concatenate([ea, jnp.zeros((pad,), jnp.float32)])
    src3 = src.reshape(NW, nch, CH)
    dst3 = dst.reshape(NW, nch, CH)
    ea2 = ea.reshape(NW, nch * CH)

    Y, R = _lin2(X, W_rel, W_root, bn=1000)
    t_part = _sc_scatter(Y, src3, dst3, ea2, nch)[:, :n, :]
    s_sm, ss_part = _mlp(t_part, R, b_rel, W1, b1, W2, b2, W3, b3, W4, b4,
                         bn=1000)
    nd = _sc_edge_losses(s_sm, src.reshape(NW, nch * CH),
                         dst.reshape(NW, nch * CH), ea2, nch)
    mc, orl = _combine(nd, ss_part)
    return (s_sm[None], mc.reshape(()), orl.reshape(()))


# speedup vs baseline: 9.0789x; 1.0409x over previous
import functools

import jax
import jax.numpy as jnp
from jax import lax
from jax.experimental import pallas as pl
from jax.experimental.pallas import tpu as pltpu
from jax.experimental.pallas import tpu_sc as plsc

NC = 2
NS = 16
NW = NC * NS
LANES = 16
CH = 100


def _lin2_body(x_ref, wa_ref, wb_ref, ya_ref, yb_ref):
    x = x_ref[...]
    ya_ref[...] = jnp.dot(x, wa_ref[...], preferred_element_type=jnp.float32)
    yb_ref[...] = jnp.dot(x, wb_ref[...], preferred_element_type=jnp.float32)


def _lin2(X, W_rel, W_root, bn):
    n, fin = X.shape
    h = W_rel.shape[1]
    grid = n // bn
    return pl.pallas_call(
        _lin2_body,
        grid=(grid,),
        in_specs=[
            pl.BlockSpec((bn, fin), lambda i: (i, 0)),
            pl.BlockSpec((fin, h), lambda i: (0, 0)),
            pl.BlockSpec((fin, h), lambda i: (0, 0)),
        ],
        out_specs=[
            pl.BlockSpec((bn, h), lambda i: (i, 0)),
            pl.BlockSpec((bn, h), lambda i: (i, 0)),
        ],
        out_shape=[
            jax.ShapeDtypeStruct((n, h), jnp.float32),
            jax.ShapeDtypeStruct((n, h), jnp.float32),
        ],
    )(X, W_rel, W_root)


def _sc_scatter(Y, src3, dst3, ea2, nch):
    n, h = Y.shape
    npad = -(-n // (NS * 8)) * (NS * 8)
    rpt = npad // NS
    zch = rpt // 5
    mesh = plsc.VectorSubcoreMesh(
        core_axis_name="c", subcore_axis_name="s", num_cores=NC, num_subcores=NS
    )

    @functools.partial(
        pl.kernel,
        out_type=jax.ShapeDtypeStruct((NC, npad, h), jnp.float32),
        mesh=mesh,
        compiler_params=pltpu.CompilerParams(needs_layout_passes=False, use_tc_tiling_on_sc=False),
        scratch_types=[
            pltpu.VMEM((nch, CH), jnp.int32),
            pltpu.VMEM((nch, CH), jnp.int32),
            pltpu.VMEM((nch * CH,), jnp.float32),
            pltpu.VMEM((CH, h), jnp.float32),
            pltpu.VMEM((CH, h), jnp.float32),
            pltpu.VMEM((zch, h), jnp.float32),
            pltpu.VMEM_SHARED((npad, h), jnp.float32),
            pltpu.SemaphoreType.DMA,
            pltpu.SemaphoreType.DMA,
        ],
    )
    def run(y_hbm, src_hbm, dst_hbm, ea_hbm, out_hbm,
            src_v, dst_v, ea_v, ra_v, rb_v, z_v, t_sh,
            ga_s, gb_s):
        cid = lax.axis_index("c")
        sid = lax.axis_index("s")
        wid = sid * NC + cid
        pltpu.sync_copy(src_hbm.at[wid], src_v)
        pltpu.sync_copy(dst_hbm.at[wid], dst_v)
        pltpu.sync_copy(ea_hbm.at[wid], ea_v)

        pltpu.async_copy(y_hbm.at[src_v.at[0]], ra_v, ga_s)
        pltpu.async_copy(y_hbm.at[src_v.at[1]], rb_v, gb_s)

        zero = jnp.zeros((LANES,), jnp.float32)
        for r in range(zch):
            for c0 in range(h // LANES):
                z_v[r, pl.ds(c0 * LANES, LANES)] = zero
        for k in range(5):
            pltpu.sync_copy(z_v, t_sh.at[pl.ds(sid * rpt + k * zch, zch)])
        plsc.subcore_barrier()

        def scale(buf, base):
            for e in range(CH):
                eab = plsc.load_gather(
                    ea_v, [jnp.full((LANES,), base + e, jnp.int32)])
                for c0 in range(h // LANES):
                    sl = pl.ds(c0 * LANES, LANES)
                    buf[e, sl] = buf[e, sl] * eab

        def pair(k, carry):
            j0 = 2 * k
            j1 = j0 + 1
            jn0 = jnp.minimum(j0 + 2, nch - 1)
            jn1 = jnp.minimum(j0 + 3, nch - 1)
            pltpu.make_async_copy(y_hbm.at[src_v.at[j0]], ra_v, ga_s).wait()
            scale(ra_v, j0 * CH)
            pltpu.sync_copy(ra_v, t_sh.at[dst_v.at[j0]], add=True)
            pltpu.async_copy(y_hbm.at[src_v.at[jn0]], ra_v, ga_s)
            pltpu.make_async_copy(y_hbm.at[src_v.at[j1]], rb_v, gb_s).wait()
            scale(rb_v, j1 * CH)
            pltpu.sync_copy(rb_v, t_sh.at[dst_v.at[j1]], add=True)
            pltpu.async_copy(y_hbm.at[src_v.at[jn1]], rb_v, gb_s)
            return carry

        lax.fori_loop(0, nch // 2, pair, 0)
        pltpu.make_async_copy(y_hbm.at[src_v.at[nch - 1]], ra_v, ga_s).wait()
        pltpu.make_async_copy(y_hbm.at[src_v.at[nch - 1]], rb_v, gb_s).wait()
        plsc.subcore_barrier()
        pltpu.sync_copy(t_sh.at[pl.ds(sid * rpt, rpt)],
                        out_hbm.at[cid, pl.ds(sid * rpt, rpt)])

    return run(Y, src3, dst3, ea2)


def _mlp_body(tp_ref, r_ref, brel_ref, w1_ref, b1_ref, w2_ref, b2_ref,
              w3_ref, b3_ref, w4_ref, b4_ref, s_ref, ss_ref):
    h = tp_ref[0] + tp_ref[1] + r_ref[...] + brel_ref[...]
    z = jnp.maximum(
        jnp.dot(h, w1_ref[...], preferred_element_type=jnp.float32)
        + b1_ref[...], 0.0)
    z = jnp.maximum(
        jnp.dot(z, w2_ref[...], preferred_element_type=jnp.float32)
        + b2_ref[...], 0.0)
    z = jnp.maximum(
        jnp.dot(z, w3_ref[...], preferred_element_type=jnp.float32)
        + b3_ref[...], 0.0)
    s = jnp.dot(z, w4_ref[...], preferred_element_type=jnp.float32) + b4_ref[...]
    m = jnp.max(s, axis=-1, keepdims=True)
    e = jnp.exp(s - m)
    sm = e / jnp.sum(e, axis=-1, keepdims=True)
    s_ref[...] = sm
    ss_ref[...] = lax.dot_general(
        sm, sm, (((0,), (0,)), ((), ())),
        preferred_element_type=jnp.float32)[None]


def _mlp(t_part, R, b_rel, W1, b1, W2, b2, W3, b3, W4, b4, bn):
    n, h = R.shape
    c = W4.shape[1]
    grid = n // bn
    wfull = lambda shp: pl.BlockSpec(shp, lambda i: tuple(0 for _ in shp))
    return pl.pallas_call(
        _mlp_body,
        grid=(grid,),
        in_specs=[
            pl.BlockSpec((NC, bn, h), lambda i: (0, i, 0)),
            pl.BlockSpec((bn, h), lambda i: (i, 0)),
            wfull((1, h)),
            wfull(W1.shape), wfull((1, W1.shape[1])),
            wfull(W2.shape), wfull((1, W2.shape[1])),
            wfull(W3.shape), wfull((1, W3.shape[1])),
            wfull(W4.shape), wfull((1, W4.shape[1])),
        ],
        out_specs=[
            pl.BlockSpec((bn, c), lambda i: (i, 0)),
            pl.BlockSpec((1, c, c), lambda i: (i, 0, 0)),
        ],
        out_shape=[
            jax.ShapeDtypeStruct((n, c), jnp.float32),
            jax.ShapeDtypeStruct((grid, c, c), jnp.float32),
        ],
    )(t_part, R, b_rel.reshape(1, h), W1, b1.reshape(1, -1), W2,
      b2.reshape(1, -1), W3, b3.reshape(1, -1), W4, b4.reshape(1, -1))


def _sc_edge_losses(s_sm, src2, dst2, ea2, nch):
    n, c = s_sm.shape
    mesh = plsc.VectorSubcoreMesh(
        core_axis_name="c", subcore_axis_name="s", num_cores=NC, num_subcores=NS
    )
    epw = nch * CH
    per_vreg = LANES // c

    @functools.partial(
        pl.kernel,
        out_type=jax.ShapeDtypeStruct((NW, 2 * LANES), jnp.float32),
        mesh=mesh,
        compiler_params=pltpu.CompilerParams(needs_layout_passes=False, use_tc_tiling_on_sc=False),
        scratch_types=[
            pltpu.VMEM((n, c), jnp.float32),
            pltpu.VMEM((epw,), jnp.int32),
            pltpu.VMEM((epw,), jnp.int32),
            pltpu.VMEM((epw,), jnp.float32),
            pltpu.VMEM((2 * LANES,), jnp.float32),
            pltpu.SemaphoreType.DMA,
        ],
    )
    def run(s_hbm, src_hbm, dst_hbm, ea_hbm, out_hbm,
            s_v, src_v, dst_v, ea_v, buf_v, sem):
        cid = lax.axis_index("c")
        sid = lax.axis_index("s")
        wid = sid * NC + cid
        pltpu.sync_copy(s_hbm, s_v)
        pltpu.sync_copy(src_hbm.at[wid], src_v)
        pltpu.sync_copy(dst_hbm.at[wid], dst_v)
        pltpu.sync_copy(ea_hbm.at[wid], ea_v)

        lane = lax.iota(jnp.int32, LANES)
        roff = lane // c
        col = lane % c

        def chunk(j, acc):
            accn, accd = acc
            for v in range(CH // per_vreg):
                idx = jnp.full((LANES,), j * CH + v * per_vreg, jnp.int32) + roff
                srow = plsc.load_gather(src_v, [idx])
                drow = plsc.load_gather(dst_v, [idx])
                eav = plsc.load_gather(ea_v, [idx])
                sv = plsc.load_gather(s_v, [srow, col])
                dv = plsc.load_gather(s_v, [drow, col])
                es = eav * sv
                accn = accn + es * dv
                accd = accd + es * sv
            return (accn, accd)

        z = jnp.zeros((LANES,), jnp.float32)
        accn, accd = lax.fori_loop(0, nch, chunk, (z, z))
        buf_v[pl.ds(0, LANES)] = accn
        buf_v[pl.ds(LANES, LANES)] = accd
        pltpu.sync_copy(buf_v, out_hbm.at[wid])

    return run(s_sm, src2, dst2, ea2)


def _combine_body(nd_ref, ss_ref, mc_ref, or_ref):
    nd = nd_ref[...]
    num = jnp.sum(nd[:, :LANES])
    den = jnp.sum(nd[:, LANES:])
    mc_ref[...] = jnp.reshape(-(num / den), (1, 1))
    ss = jnp.sum(ss_ref[...], axis=0)
    k = ss.shape[0]
    ssn = jnp.sqrt(jnp.sum(ss * ss))
    eye = (lax.broadcasted_iota(jnp.int32, (k, k), 0)
           == lax.broadcasted_iota(jnp.int32, (k, k), 1)).astype(jnp.float32)
    diff = ss / ssn - eye / jnp.sqrt(jnp.float32(k))
    or_ref[...] = jnp.reshape(jnp.sqrt(jnp.sum(diff * diff)), (1, 1))


def _combine(nd, ss_part):
    g, c, _ = ss_part.shape
    return pl.pallas_call(
        _combine_body,
        in_specs=[
            pl.BlockSpec(nd.shape, lambda: (0, 0)),
            pl.BlockSpec(ss_part.shape, lambda: (0, 0, 0)),
        ],
        out_specs=[
            pl.BlockSpec((1, 1), lambda: (0, 0)),
            pl.BlockSpec((1, 1), lambda: (0, 0)),
        ],
        out_shape=[
            jax.ShapeDtypeStruct((1, 1), jnp.float32),
            jax.ShapeDtypeStruct((1, 1), jnp.float32),
        ],
    )(nd, ss_part)


def kernel(X, edge_index, edge_attr, W_root, W_rel, b_rel,
           W1, b1, W2, b2, W3, b3, W4, b4):
    n, fin = X.shape
    e = edge_attr.shape[0]
    h = W_rel.shape[1]

    nch = -(-e // (NW * CH))
    nch += nch % 2
    epad = NW * CH * nch
    src = edge_index[0]
    dst = edge_index[1]
    ea = edge_attr.astype(jnp.float32)
    if epad != e:
        pad = epad - e
        zi = jnp.zeros((pad,), jnp.int32)
        src = jnp.concatenate([src, zi])
        dst = jnp.concatenate([dst, zi])
        ea = jnp.concatenate([ea, jnp.zeros((pad,), jnp.float32)])
    src3 = src.reshape(NW, nch, CH)
    dst3 = dst.reshape(NW, nch, CH)
    ea2 = ea.reshape(NW, nch * CH)

    Y, R = _lin2(X, W_rel, W_root, bn=1000)
    t_part = _sc_scatter(Y, src3, dst3, ea2, nch)[:, :n, :]
    s_sm, ss_part = _mlp(t_part, R, b_rel, W1, b1, W2, b2, W3, b3, W4, b4,
                         bn=1000)
    nd = _sc_edge_losses(s_sm, src.reshape(NW, nch * CH),
                         dst.reshape(NW, nch * CH), ea2, nch)
    mc, orl = _combine(nd, ss_part)
    return (s_sm[None], mc.reshape(()), orl.reshape(()))
